# Initial kernel scaffold; baseline (speedup 1.0000x reference)
import jax
import jax.numpy as jnp
from jax.experimental import pallas as pl


def kernel(token_ids, weight):
    raise NotImplementedError("write your pallas kernel here")


# speedup vs baseline: 5.7598x; 5.7598x over previous
import functools

import jax
import jax.numpy as jnp
from jax import lax
from jax.experimental import pallas as pl
from jax.experimental.pallas import tpu as pltpu
from jax.experimental.pallas import tpu_sc as plsc

D_MODEL = 128
NC = 2
NS = 16
NW = NC * NS
CHUNK = 128


@functools.partial(jax.jit, static_argnames=("batch",))
def _emb_lookup(table, idx_flat, *, batch):
    b_per_w = batch // NW
    n_chunks = b_per_w // CHUNK
    mesh = plsc.VectorSubcoreMesh(
        core_axis_name="c", subcore_axis_name="s",
        num_cores=NC, num_subcores=NS)

    @functools.partial(
        pl.kernel,
        out_type=jax.ShapeDtypeStruct((batch, D_MODEL), jnp.float32),
        mesh=mesh,
        scratch_types=[
            pltpu.VMEM((b_per_w,), jnp.int32),
            pltpu.VMEM((CHUNK, D_MODEL), jnp.float32),
            pltpu.SemaphoreType.DMA,
        ],
    )
    def emb_kernel(table_hbm, idx_hbm, out_hbm, idx_v, rows_v, gsem):
        wid = lax.axis_index("s") * NC + lax.axis_index("c")
        base = wid * b_per_w
        pltpu.sync_copy(idx_hbm.at[pl.ds(base, b_per_w)], idx_v)

        def body(c, carry):
            off = c * CHUNK
            pltpu.async_copy(
                table_hbm.at[idx_v.at[pl.ds(off, CHUNK)]], rows_v, gsem
            ).wait()
            pltpu.sync_copy(rows_v, out_hbm.at[pl.ds(base + off, CHUNK)])
            return carry

        lax.fori_loop(0, n_chunks, body, 0)

    return emb_kernel(table, idx_flat)


def kernel(token_ids, weight):
    b, t = token_ids.shape
    idx_flat = token_ids.reshape(b * t).astype(jnp.int32)
    out = _emb_lookup(weight, idx_flat, batch=b * t)
    return out.reshape(b, t, D_MODEL)


# speedup vs baseline: 7.8889x; 1.3696x over previous
import functools

import jax
import jax.numpy as jnp
from jax import lax
from jax.experimental import pallas as pl
from jax.experimental.pallas import tpu as pltpu
from jax.experimental.pallas import tpu_sc as plsc

D_MODEL = 128
NC = 2
NS = 16
NW = NC * NS
CHUNK = 128


@functools.partial(jax.jit, static_argnames=("batch",))
def _emb_lookup(table, idx_flat, *, batch):
    b_per_w = batch // NW
    n_chunks = b_per_w // CHUNK
    assert n_chunks % 2 == 0 and n_chunks >= 4
    mesh = plsc.VectorSubcoreMesh(
        core_axis_name="c", subcore_axis_name="s",
        num_cores=NC, num_subcores=NS)

    @functools.partial(
        pl.kernel,
        out_type=jax.ShapeDtypeStruct((batch, D_MODEL), jnp.float32),
        mesh=mesh,
        scratch_types=[
            pltpu.VMEM((b_per_w,), jnp.int32),
            pltpu.VMEM((2, CHUNK, D_MODEL), jnp.float32),
            pltpu.SemaphoreType.DMA,
            pltpu.SemaphoreType.DMA,
        ],
    )
    def emb_kernel(table_hbm, idx_hbm, out_hbm, idx_v, rows_v, sem0, sem1):
        wid = lax.axis_index("s") * NC + lax.axis_index("c")
        base = wid * b_per_w
        pltpu.sync_copy(idx_hbm.at[pl.ds(base, b_per_w)], idx_v)
        sems = (sem0, sem1)

        def fire(c, b):
            pltpu.async_copy(
                table_hbm.at[idx_v.at[pl.ds(c * CHUNK, CHUNK)]],
                rows_v.at[b], sems[b])

        def drain(b):
            pltpu.make_async_copy(
                table_hbm.at[idx_v.at[pl.ds(0, CHUNK)]],
                rows_v.at[b], sems[b]).wait()

        def write(c, b):
            pltpu.sync_copy(
                rows_v.at[b], out_hbm.at[pl.ds(base + c * CHUNK, CHUNK)])

        fire(0, 0)
        fire(1, 1)
        drain(0)
        write(0, 0)

        def body(i, carry):
            c0 = 2 + 2 * i
            fire(c0, 0)
            drain(1)
            write(c0 - 1, 1)
            fire(c0 + 1, 1)
            drain(0)
            write(c0, 0)
            return carry

        lax.fori_loop(0, (n_chunks - 2) // 2, body, 0)
        drain(1)
        write(n_chunks - 1, 1)

    return emb_kernel(table, idx_flat)


def kernel(token_ids, weight):
    b, t = token_ids.shape
    idx_flat = token_ids.reshape(b * t).astype(jnp.int32)
    out = _emb_lookup(weight, idx_flat, batch=b * t)
    return out.reshape(b, t, D_MODEL)


# speedup vs baseline: 7.9558x; 1.0085x over previous
import functools

import jax
import jax.numpy as jnp
from jax import lax
from jax.experimental import pallas as pl
from jax.experimental.pallas import tpu as pltpu
from jax.experimental.pallas import tpu_sc as plsc

D_MODEL = 128
NC = 2
NS = 16
NW = NC * NS
CHUNK = 128
NBUF = 4


@functools.partial(jax.jit, static_argnames=("batch",))
def _emb_lookup(table, idx_flat, *, batch):
    b_per_w = batch // NW
    n_chunks = b_per_w // CHUNK
    assert (n_chunks - 6) % 4 == 0
    mesh = plsc.VectorSubcoreMesh(
        core_axis_name="c", subcore_axis_name="s",
        num_cores=NC, num_subcores=NS)

    @functools.partial(
        pl.kernel,
        out_type=jax.ShapeDtypeStruct((batch, D_MODEL), jnp.float32),
        mesh=mesh,
        scratch_types=[
            pltpu.VMEM((b_per_w,), jnp.int32),
            pltpu.VMEM((NBUF, CHUNK, D_MODEL), jnp.float32),
            [pltpu.SemaphoreType.DMA] * NBUF,
            [pltpu.SemaphoreType.DMA] * NBUF,
        ],
    )
    def emb_kernel(table_hbm, idx_hbm, out_hbm, idx_v, rows_v, gsems, wsems):
        wid = lax.axis_index("s") * NC + lax.axis_index("c")
        base = wid * b_per_w
        pltpu.sync_copy(idx_hbm.at[pl.ds(base, b_per_w)], idx_v)

        def fire_gather(c, s):
            pltpu.async_copy(
                table_hbm.at[idx_v.at[pl.ds(c * CHUNK, CHUNK)]],
                rows_v.at[s], gsems[s])

        def drain_gather(s):
            pltpu.make_async_copy(
                table_hbm.at[idx_v.at[pl.ds(0, CHUNK)]],
                rows_v.at[s], gsems[s]).wait()

        def fire_write(c, s):
            pltpu.async_copy(
                rows_v.at[s], out_hbm.at[pl.ds(base + c * CHUNK, CHUNK)],
                wsems[s])

        def drain_write(s):
            pltpu.make_async_copy(
                rows_v.at[s], out_hbm.at[pl.ds(base, CHUNK)],
                wsems[s]).wait()

        def step(c, s, drain_w, fire_next):
            drain_gather(s)
            fire_write(c, s)
            if drain_w:
                drain_write((s + 2) % NBUF)
            if fire_next:
                fire_gather(c + 2, (s + 2) % NBUF)

        fire_gather(0, 0)
        fire_gather(1, 1)
        step(0, 0, False, True)
        step(1, 1, False, True)

        def body(i, carry):
            c0 = 2 + 4 * i
            for k in range(4):
                step(c0 + k, (2 + k) % NBUF, True, True)
            return carry

        lax.fori_loop(0, (n_chunks - 6) // 4, body, 0)
        step(n_chunks - 4, (n_chunks - 4) % NBUF, True, True)
        step(n_chunks - 3, (n_chunks - 3) % NBUF, True, True)
        step(n_chunks - 2, (n_chunks - 2) % NBUF, True, False)
        step(n_chunks - 1, (n_chunks - 1) % NBUF, True, False)
        drain_write((n_chunks - 2) % NBUF)
        drain_write((n_chunks - 1) % NBUF)

    return emb_kernel(table, idx_flat)


def kernel(token_ids, weight):
    b, t = token_ids.shape
    idx_flat = token_ids.reshape(b * t).astype(jnp.int32)
    out = _emb_lookup(weight, idx_flat, batch=b * t)
    return out.reshape(b, t, D_MODEL)
